# baseline (speedup 1.0000x reference)
import functools

import jax
import jax.numpy as jnp
from jax.experimental import pallas as pl

BM = 512


def _matmul_kernel(x_ref, w0_ref, w1_ref, o_ref):
    x = x_ref[...]
    o_ref[0] = jnp.dot(w0_ref[...], x, preferred_element_type=jnp.float32)
    o_ref[1] = jnp.dot(w1_ref[...], x, preferred_element_type=jnp.float32)


@functools.partial(jax.jit, static_argnames=())
def kernel(input, weight):
    m, k = weight.shape
    _, n = input.shape
    half = m // 2 // BM
    out = pl.pallas_call(
        _matmul_kernel,
        grid=(half,),
        in_specs=[
            pl.BlockSpec((k, n), lambda i: (0, 0)),
            pl.BlockSpec((BM, k), lambda i: (i, 0)),
            pl.BlockSpec((BM, k), lambda i: (half + i, 0)),
        ],
        out_specs=pl.BlockSpec((2, BM, n), lambda i: (0, i, 0)),
        out_shape=jax.ShapeDtypeStruct((2, m // 2, n), jnp.float32),
    )(input, weight, weight)
    return out.reshape(m, n)


# speedup vs baseline: 1.1953x; 1.1953x over previous
import functools

import jax
import jax.numpy as jnp
from jax.experimental import pallas as pl

BM = 512


def _matmul_kernel(x_ref, w_ref, o_ref):
    o_ref[...] = jax.lax.dot_general(
        x_ref[...],
        w_ref[...],
        (((0,), (1,)), ((), ())),
        preferred_element_type=jnp.float32,
    )


@functools.partial(jax.jit, static_argnames=())
def kernel(input, weight):
    m, k = weight.shape
    _, n = input.shape
    out_t = pl.pallas_call(
        _matmul_kernel,
        grid=(m // BM,),
        in_specs=[
            pl.BlockSpec((k, n), lambda i: (0, 0)),
            pl.BlockSpec((BM, k), lambda i: (i, 0)),
        ],
        out_specs=pl.BlockSpec((n, BM), lambda i: (0, i)),
        out_shape=jax.ShapeDtypeStruct((n, m), jnp.float32),
    )(input, weight)
    return out_t.T
